# Initial kernel scaffold; baseline (speedup 1.0000x reference)
import jax
import jax.numpy as jnp
from jax.experimental import pallas as pl


def kernel(nodes, edges, senders, receivers, W_e, b_e, epsilon, W1, b1, W2, b2):
    raise NotImplementedError("write your pallas kernel here")


# speedup vs baseline: 2.0294x; 2.0294x over previous
import functools

import jax
import jax.numpy as jnp
from jax import lax
from jax.experimental import pallas as pl
from jax.experimental.pallas import tpu as pltpu
from jax.experimental.pallas import tpu_sc as plsc

N_NODES = 10000
N_EDGES = 160000
D_FEAT = 256
D_HALF = 128
D_EDGE = 16
D_HID = 1024

E_BLK = 128
N_SUBCORES = 16
N_CORES = 2
N_WORKERS = N_CORES * N_SUBCORES


def _mish(x):
    return x * jnp.tanh(jax.nn.softplus(x))


def _sc_gather(nodes, senders):
    nblocks = N_EDGES // E_BLK
    mesh = plsc.VectorSubcoreMesh(core_axis_name="c", subcore_axis_name="s")

    @functools.partial(
        pl.kernel,
        out_type=jax.ShapeDtypeStruct((N_EDGES, D_FEAT), jnp.float32),
        mesh=mesh,
        scratch_types=[
            pltpu.VMEM((E_BLK,), jnp.int32),
            pltpu.VMEM((E_BLK, D_FEAT), jnp.float32),
            pltpu.SemaphoreType.DMA,
        ],
    )
    def k(nodes_hbm, send_hbm, out_hbm, idx_v, rows_v, sem):
        wid = lax.axis_index("s") * N_CORES + lax.axis_index("c")

        @pl.loop(wid, nblocks, step=N_WORKERS)
        def _(b):
            base = b * E_BLK
            pltpu.sync_copy(send_hbm.at[pl.ds(base, E_BLK)], idx_v)
            pltpu.async_copy(nodes_hbm.at[idx_v], rows_v, sem).wait()
            pltpu.sync_copy(rows_v, out_hbm.at[pl.ds(base, E_BLK)])

    return k(nodes, senders)


def _tc_messages(sent, edges, W_e, b_e):
    BLK = 1000
    grid = (N_EDGES // BLK,)

    def body(sent_ref, edges_ref, we_ref, be_ref, lo_ref, hi_ref):
        emb = jnp.dot(edges_ref[...], we_ref[...],
                      preferred_element_type=jnp.float32)
        m = _mish(sent_ref[...] + emb + be_ref[...])
        lo_ref[...] = m[:, :D_HALF]
        hi_ref[...] = m[:, D_HALF:]

    return pl.pallas_call(
        body,
        grid=grid,
        in_specs=[
            pl.BlockSpec((BLK, D_FEAT), lambda i: (i, 0)),
            pl.BlockSpec((BLK, D_EDGE), lambda i: (i, 0)),
            pl.BlockSpec((D_EDGE, D_FEAT), lambda i: (0, 0)),
            pl.BlockSpec((1, D_FEAT), lambda i: (0, 0)),
        ],
        out_specs=[
            pl.BlockSpec((BLK, D_HALF), lambda i: (i, 0)),
            pl.BlockSpec((BLK, D_HALF), lambda i: (i, 0)),
        ],
        out_shape=[
            jax.ShapeDtypeStruct((N_EDGES, D_HALF), jnp.float32),
            jax.ShapeDtypeStruct((N_EDGES, D_HALF), jnp.float32),
        ],
    )(sent, edges, W_e, b_e.reshape(1, D_FEAT))


def _sc_scatter(mlo, mhi, receivers):
    nblocks = N_EDGES // E_BLK
    ROW_BLK = 80
    n_row_blks = N_NODES // ROW_BLK
    mesh = plsc.VectorSubcoreMesh(core_axis_name="c", subcore_axis_name="s")

    @functools.partial(
        pl.kernel,
        out_type=(
            jax.ShapeDtypeStruct((N_NODES, D_HALF), jnp.float32),
            jax.ShapeDtypeStruct((N_NODES, D_HALF), jnp.float32),
        ),
        mesh=mesh,
        scratch_types=[
            pltpu.VMEM((E_BLK,), jnp.int32),
            pltpu.VMEM((E_BLK, D_HALF), jnp.float32),
            pltpu.VMEM_SHARED((N_NODES, D_HALF), jnp.float32),
        ],
    )
    def k(mlo_hbm, mhi_hbm, recv_hbm, olo_hbm, ohi_hbm, idx_v, msg_v, acc):
        cid = lax.axis_index("c")
        sid = lax.axis_index("s")

        def halfwork(m_hbm, o_hbm):
            @pl.loop(0, ROW_BLK)
            def _(r):
                @pl.loop(0, D_HALF, step=16)
                def _(cc):
                    msg_v.at[pl.ds(r, 1), pl.ds(cc, 16)][...] = (
                        jnp.zeros((1, 16), jnp.float32))

            @pl.loop(sid, n_row_blks, step=N_SUBCORES)
            def _(t):
                pltpu.sync_copy(msg_v.at[pl.ds(0, ROW_BLK)],
                                acc.at[pl.ds(t * ROW_BLK, ROW_BLK)])

            plsc.subcore_barrier()

            @pl.loop(sid, nblocks, step=N_SUBCORES)
            def _(b):
                base = b * E_BLK
                pltpu.sync_copy(recv_hbm.at[pl.ds(base, E_BLK)], idx_v)
                pltpu.sync_copy(m_hbm.at[pl.ds(base, E_BLK)], msg_v)
                pltpu.sync_copy(msg_v, acc.at[idx_v], add=True)

            plsc.subcore_barrier()

            @pl.loop(sid, n_row_blks, step=N_SUBCORES)
            def _(t):
                pltpu.sync_copy(acc.at[pl.ds(t * ROW_BLK, ROW_BLK)],
                                o_hbm.at[pl.ds(t * ROW_BLK, ROW_BLK)])

        @pl.when(cid == 0)
        def _():
            halfwork(mlo_hbm, olo_hbm)

        @pl.when(cid == 1)
        def _():
            halfwork(mhi_hbm, ohi_hbm)

    return k(mlo, mhi, receivers)


def _tc_mlp(nodes, rlo, rhi, epsilon, W1, b1, W2, b2):
    BLK = 1000
    grid = (N_NODES // BLK,)

    def body(nodes_ref, rlo_ref, rhi_ref, eps_ref, w1_ref, b1_ref, w2_ref,
             b2_ref, o_ref):
        received = jnp.concatenate([rlo_ref[...], rhi_ref[...]], axis=1)
        h = (1.0 + eps_ref[...]) * nodes_ref[...] + received
        a = _mish(jnp.dot(h, w1_ref[...],
                          preferred_element_type=jnp.float32) + b1_ref[...])
        o_ref[...] = jnp.dot(a, w2_ref[...],
                             preferred_element_type=jnp.float32) + b2_ref[...]

    return pl.pallas_call(
        body,
        grid=grid,
        in_specs=[
            pl.BlockSpec((BLK, D_FEAT), lambda i: (i, 0)),
            pl.BlockSpec((BLK, D_HALF), lambda i: (i, 0)),
            pl.BlockSpec((BLK, D_HALF), lambda i: (i, 0)),
            pl.BlockSpec((1, 1), lambda i: (0, 0)),
            pl.BlockSpec((D_FEAT, D_HID), lambda i: (0, 0)),
            pl.BlockSpec((1, D_HID), lambda i: (0, 0)),
            pl.BlockSpec((D_HID, D_FEAT), lambda i: (0, 0)),
            pl.BlockSpec((1, D_FEAT), lambda i: (0, 0)),
        ],
        out_specs=pl.BlockSpec((BLK, D_FEAT), lambda i: (i, 0)),
        out_shape=jax.ShapeDtypeStruct((N_NODES, D_FEAT), jnp.float32),
    )(nodes, rlo, rhi, epsilon, W1, b1.reshape(1, D_HID), W2,
      b2.reshape(1, D_FEAT))


def kernel(nodes, edges, senders, receivers, W_e, b_e, epsilon, W1, b1, W2, b2):
    sent = _sc_gather(nodes, senders)
    mlo, mhi = _tc_messages(sent, edges, W_e, b_e)
    rlo, rhi = _sc_scatter(mlo, mhi, receivers)
    return _tc_mlp(nodes, rlo, rhi, epsilon, W1, b1, W2, b2)


# speedup vs baseline: 2.5252x; 1.2443x over previous
import functools

import jax
import jax.numpy as jnp
from jax import lax
from jax.experimental import pallas as pl
from jax.experimental.pallas import tpu as pltpu
from jax.experimental.pallas import tpu_sc as plsc

N_NODES = 10000
N_EDGES = 160000
D_FEAT = 256
D_HALF = 128
D_EDGE = 16
D_HID = 1024

E_BLK = 128
N_SUBCORES = 16
N_CORES = 2
N_WORKERS = N_CORES * N_SUBCORES


def _mish(x):
    return x * jnp.tanh(jax.nn.softplus(x))


def _sc_gather(nodes, senders):
    nblocks = N_EDGES // E_BLK
    mesh = plsc.VectorSubcoreMesh(core_axis_name="c", subcore_axis_name="s")

    @functools.partial(
        pl.kernel,
        out_type=jax.ShapeDtypeStruct((N_EDGES, D_FEAT), jnp.float32),
        mesh=mesh,
    )
    def k(nodes_hbm, idx_hbm, out_hbm):
        def body(i_vmem, o_vmem):
            pltpu.sync_copy(nodes_hbm.at[i_vmem.at[0]], o_vmem)

        pltpu.emit_pipeline(
            body,
            grid=(nblocks,),
            in_specs=[pl.BlockSpec((1, E_BLK), lambda i: (0, i))],
            out_specs=[pl.BlockSpec((E_BLK, D_FEAT), lambda i: (i, 0))],
            core_axis_name=("c", "s"),
            dimension_semantics=(pltpu.PARALLEL,),
        )(idx_hbm, out_hbm)

    return k(nodes, senders.reshape(1, N_EDGES))


def _tc_messages(sent, edges, W_e, b_e):
    BLK = 1000
    grid = (N_EDGES // BLK,)

    def body(sent_ref, edges_ref, we_ref, be_ref, lo_ref, hi_ref):
        emb = jnp.dot(edges_ref[...], we_ref[...],
                      preferred_element_type=jnp.float32)
        m = _mish(sent_ref[...] + emb + be_ref[...])
        lo_ref[...] = m[:, :D_HALF]
        hi_ref[...] = m[:, D_HALF:]

    return pl.pallas_call(
        body,
        grid=grid,
        in_specs=[
            pl.BlockSpec((BLK, D_FEAT), lambda i: (i, 0)),
            pl.BlockSpec((BLK, D_EDGE), lambda i: (i, 0)),
            pl.BlockSpec((D_EDGE, D_FEAT), lambda i: (0, 0)),
            pl.BlockSpec((1, D_FEAT), lambda i: (0, 0)),
        ],
        out_specs=[
            pl.BlockSpec((BLK, D_HALF), lambda i: (i, 0)),
            pl.BlockSpec((BLK, D_HALF), lambda i: (i, 0)),
        ],
        out_shape=[
            jax.ShapeDtypeStruct((N_EDGES, D_HALF), jnp.float32),
            jax.ShapeDtypeStruct((N_EDGES, D_HALF), jnp.float32),
        ],
    )(sent, edges, W_e, b_e.reshape(1, D_FEAT))


def _sc_scatter(mlo, mhi, receivers):
    nblocks = N_EDGES // E_BLK
    ROW_BLK = 80
    n_row_blks = N_NODES // ROW_BLK
    mesh = plsc.VectorSubcoreMesh(core_axis_name="c", subcore_axis_name="s")

    @functools.partial(
        pl.kernel,
        out_type=(
            jax.ShapeDtypeStruct((N_NODES, D_HALF), jnp.float32),
            jax.ShapeDtypeStruct((N_NODES, D_HALF), jnp.float32),
        ),
        mesh=mesh,
        scratch_types=[
            pltpu.VMEM((ROW_BLK, D_HALF), jnp.float32),
            pltpu.VMEM_SHARED((N_NODES, D_HALF), jnp.float32),
        ],
    )
    def k(mlo_hbm, mhi_hbm, recv_hbm, olo_hbm, ohi_hbm, zbuf, acc):
        cid = lax.axis_index("c")
        sid = lax.axis_index("s")

        @pl.loop(0, ROW_BLK)
        def _(r):
            @pl.loop(0, D_HALF, step=16)
            def _(cc):
                zbuf.at[pl.ds(r, 1), pl.ds(cc, 16)][...] = (
                    jnp.zeros((1, 16), jnp.float32))

        @pl.loop(sid, n_row_blks, step=N_SUBCORES)
        def _(t):
            pltpu.sync_copy(zbuf, acc.at[pl.ds(t * ROW_BLK, ROW_BLK)])

        plsc.subcore_barrier()

        def halfwork(m_hbm, o_hbm):
            def body(i_vmem, m_vmem):
                pltpu.sync_copy(m_vmem, acc.at[i_vmem.at[0]], add=True)

            pltpu.emit_pipeline(
                body,
                grid=(nblocks,),
                in_specs=[
                    pl.BlockSpec((1, E_BLK), lambda i: (0, i)),
                    pl.BlockSpec((E_BLK, D_HALF), lambda i: (i, 0)),
                ],
                out_specs=[],
                core_axis_name="s",
                dimension_semantics=(pltpu.PARALLEL,),
            )(recv_hbm, m_hbm)

            plsc.subcore_barrier()

            @pl.loop(sid, n_row_blks, step=N_SUBCORES)
            def _(t):
                pltpu.sync_copy(acc.at[pl.ds(t * ROW_BLK, ROW_BLK)],
                                o_hbm.at[pl.ds(t * ROW_BLK, ROW_BLK)])

        @pl.when(cid == 0)
        def _():
            halfwork(mlo_hbm, olo_hbm)

        @pl.when(cid == 1)
        def _():
            halfwork(mhi_hbm, ohi_hbm)

    return k(mlo, mhi, receivers.reshape(1, N_EDGES))


def _tc_mlp(nodes, rlo, rhi, epsilon, W1, b1, W2, b2):
    BLK = 1000
    grid = (N_NODES // BLK,)

    def body(nodes_ref, rlo_ref, rhi_ref, eps_ref, w1_ref, b1_ref, w2_ref,
             b2_ref, o_ref):
        received = jnp.concatenate([rlo_ref[...], rhi_ref[...]], axis=1)
        h = (1.0 + eps_ref[...]) * nodes_ref[...] + received
        a = _mish(jnp.dot(h, w1_ref[...],
                          preferred_element_type=jnp.float32) + b1_ref[...])
        o_ref[...] = jnp.dot(a, w2_ref[...],
                             preferred_element_type=jnp.float32) + b2_ref[...]

    return pl.pallas_call(
        body,
        grid=grid,
        in_specs=[
            pl.BlockSpec((BLK, D_FEAT), lambda i: (i, 0)),
            pl.BlockSpec((BLK, D_HALF), lambda i: (i, 0)),
            pl.BlockSpec((BLK, D_HALF), lambda i: (i, 0)),
            pl.BlockSpec((1, 1), lambda i: (0, 0)),
            pl.BlockSpec((D_FEAT, D_HID), lambda i: (0, 0)),
            pl.BlockSpec((1, D_HID), lambda i: (0, 0)),
            pl.BlockSpec((D_HID, D_FEAT), lambda i: (0, 0)),
            pl.BlockSpec((1, D_FEAT), lambda i: (0, 0)),
        ],
        out_specs=pl.BlockSpec((BLK, D_FEAT), lambda i: (i, 0)),
        out_shape=jax.ShapeDtypeStruct((N_NODES, D_FEAT), jnp.float32),
    )(nodes, rlo, rhi, epsilon, W1, b1.reshape(1, D_HID), W2,
      b2.reshape(1, D_FEAT))


def kernel(nodes, edges, senders, receivers, W_e, b_e, epsilon, W1, b1, W2, b2):
    sent = _sc_gather(nodes, senders)
    mlo, mhi = _tc_messages(sent, edges, W_e, b_e)
    rlo, rhi = _sc_scatter(mlo, mhi, receivers)
    return _tc_mlp(nodes, rlo, rhi, epsilon, W1, b1, W2, b2)


# speedup vs baseline: 2.6937x; 1.0667x over previous
import functools

import jax
import jax.numpy as jnp
from jax import lax
from jax.experimental import pallas as pl
from jax.experimental.pallas import tpu as pltpu
from jax.experimental.pallas import tpu_sc as plsc

N_NODES = 10000
N_EDGES = 160000
D_FEAT = 256
D_HALF = 128
D_EDGE = 16
D_HID = 1024

E_BLK = 128
N_SUBCORES = 16
N_CORES = 2
N_WORKERS = N_CORES * N_SUBCORES


def _mish(x):
    u = 1.0 + jnp.exp(jnp.minimum(x, 20.0))
    uu = u * u
    return x * ((uu - 1.0) / (uu + 1.0))


def _sc_gather(nodes, senders):
    nblocks = N_EDGES // E_BLK
    mesh = plsc.VectorSubcoreMesh(core_axis_name="c", subcore_axis_name="s")

    @functools.partial(
        pl.kernel,
        out_type=jax.ShapeDtypeStruct((N_EDGES, D_FEAT), jnp.float32),
        mesh=mesh,
    )
    def k(nodes_hbm, idx_hbm, out_hbm):
        def body(i_vmem, o_vmem):
            pltpu.sync_copy(nodes_hbm.at[i_vmem.at[0]], o_vmem)

        pltpu.emit_pipeline(
            body,
            grid=(nblocks,),
            in_specs=[pl.BlockSpec((1, E_BLK), lambda i: (0, i))],
            out_specs=[pl.BlockSpec((E_BLK, D_FEAT), lambda i: (i, 0))],
            core_axis_name=("c", "s"),
            dimension_semantics=(pltpu.PARALLEL,),
        )(idx_hbm, out_hbm)

    return k(nodes, senders.reshape(1, N_EDGES))


def _tc_messages(sent, edges, W_e, b_e):
    BLK = 1000
    grid = (N_EDGES // BLK,)

    def body(sent_ref, edges_ref, we_ref, be_ref, lo_ref, hi_ref):
        emb = jnp.dot(edges_ref[...], we_ref[...],
                      preferred_element_type=jnp.float32)
        m = _mish(sent_ref[...] + emb + be_ref[...])
        lo_ref[...] = m[:, :D_HALF]
        hi_ref[...] = m[:, D_HALF:]

    return pl.pallas_call(
        body,
        grid=grid,
        in_specs=[
            pl.BlockSpec((BLK, D_FEAT), lambda i: (i, 0)),
            pl.BlockSpec((BLK, D_EDGE), lambda i: (i, 0)),
            pl.BlockSpec((D_EDGE, D_FEAT), lambda i: (0, 0)),
            pl.BlockSpec((1, D_FEAT), lambda i: (0, 0)),
        ],
        out_specs=[
            pl.BlockSpec((BLK, D_HALF), lambda i: (i, 0)),
            pl.BlockSpec((BLK, D_HALF), lambda i: (i, 0)),
        ],
        out_shape=[
            jax.ShapeDtypeStruct((N_EDGES, D_HALF), jnp.float32),
            jax.ShapeDtypeStruct((N_EDGES, D_HALF), jnp.float32),
        ],
    )(sent, edges, W_e, b_e.reshape(1, D_FEAT))


def _sc_scatter(mlo, mhi, receivers):
    nblocks = N_EDGES // E_BLK
    ROW_BLK = 80
    n_row_blks = N_NODES // ROW_BLK
    mesh = plsc.VectorSubcoreMesh(core_axis_name="c", subcore_axis_name="s")

    @functools.partial(
        pl.kernel,
        out_type=(
            jax.ShapeDtypeStruct((N_NODES, D_HALF), jnp.float32),
            jax.ShapeDtypeStruct((N_NODES, D_HALF), jnp.float32),
        ),
        mesh=mesh,
        scratch_types=[
            pltpu.VMEM((ROW_BLK, D_HALF), jnp.float32),
            pltpu.VMEM_SHARED((N_NODES, D_HALF), jnp.float32),
        ],
    )
    def k(mlo_hbm, mhi_hbm, recv_hbm, olo_hbm, ohi_hbm, zbuf, acc):
        cid = lax.axis_index("c")
        sid = lax.axis_index("s")

        @pl.loop(0, ROW_BLK)
        def _(r):
            @pl.loop(0, D_HALF, step=16)
            def _(cc):
                zbuf.at[pl.ds(r, 1), pl.ds(cc, 16)][...] = (
                    jnp.zeros((1, 16), jnp.float32))

        @pl.loop(sid, n_row_blks, step=N_SUBCORES)
        def _(t):
            pltpu.sync_copy(zbuf, acc.at[pl.ds(t * ROW_BLK, ROW_BLK)])

        plsc.subcore_barrier()

        def halfwork(m_hbm, o_hbm):
            def body(i_vmem, m_vmem):
                pltpu.sync_copy(m_vmem, acc.at[i_vmem.at[0]], add=True)

            pltpu.emit_pipeline(
                body,
                grid=(nblocks,),
                in_specs=[
                    pl.BlockSpec((1, E_BLK), lambda i: (0, i)),
                    pl.BlockSpec((E_BLK, D_HALF), lambda i: (i, 0)),
                ],
                out_specs=[],
                core_axis_name="s",
                dimension_semantics=(pltpu.PARALLEL,),
            )(recv_hbm, m_hbm)

            plsc.subcore_barrier()

            @pl.loop(sid, n_row_blks, step=N_SUBCORES)
            def _(t):
                pltpu.sync_copy(acc.at[pl.ds(t * ROW_BLK, ROW_BLK)],
                                o_hbm.at[pl.ds(t * ROW_BLK, ROW_BLK)])

        @pl.when(cid == 0)
        def _():
            halfwork(mlo_hbm, olo_hbm)

        @pl.when(cid == 1)
        def _():
            halfwork(mhi_hbm, ohi_hbm)

    return k(mlo, mhi, receivers.reshape(1, N_EDGES))


def _tc_mlp(nodes, rlo, rhi, epsilon, W1, b1, W2, b2):
    BLK = 1000
    grid = (N_NODES // BLK,)

    def body(nodes_ref, rlo_ref, rhi_ref, eps_ref, w1_ref, b1_ref, w2_ref,
             b2_ref, o_ref):
        received = jnp.concatenate([rlo_ref[...], rhi_ref[...]], axis=1)
        h = (1.0 + eps_ref[...]) * nodes_ref[...] + received
        a = _mish(jnp.dot(h, w1_ref[...],
                          preferred_element_type=jnp.float32) + b1_ref[...])
        o_ref[...] = jnp.dot(a, w2_ref[...],
                             preferred_element_type=jnp.float32) + b2_ref[...]

    return pl.pallas_call(
        body,
        grid=grid,
        in_specs=[
            pl.BlockSpec((BLK, D_FEAT), lambda i: (i, 0)),
            pl.BlockSpec((BLK, D_HALF), lambda i: (i, 0)),
            pl.BlockSpec((BLK, D_HALF), lambda i: (i, 0)),
            pl.BlockSpec((1, 1), lambda i: (0, 0)),
            pl.BlockSpec((D_FEAT, D_HID), lambda i: (0, 0)),
            pl.BlockSpec((1, D_HID), lambda i: (0, 0)),
            pl.BlockSpec((D_HID, D_FEAT), lambda i: (0, 0)),
            pl.BlockSpec((1, D_FEAT), lambda i: (0, 0)),
        ],
        out_specs=pl.BlockSpec((BLK, D_FEAT), lambda i: (i, 0)),
        out_shape=jax.ShapeDtypeStruct((N_NODES, D_FEAT), jnp.float32),
    )(nodes, rlo, rhi, epsilon, W1, b1.reshape(1, D_HID), W2,
      b2.reshape(1, D_FEAT))


def kernel(nodes, edges, senders, receivers, W_e, b_e, epsilon, W1, b1, W2, b2):
    sent = _sc_gather(nodes, senders)
    mlo, mhi = _tc_messages(sent, edges, W_e, b_e)
    rlo, rhi = _sc_scatter(mlo, mhi, receivers)
    return _tc_mlp(nodes, rlo, rhi, epsilon, W1, b1, W2, b2)
